# Initial kernel scaffold; baseline (speedup 1.0000x reference)
import jax
import jax.numpy as jnp
from jax.experimental import pallas as pl


def kernel(x):
    raise NotImplementedError("write your pallas kernel here")


# speedup vs baseline: 18.7410x; 18.7410x over previous
import numpy as np
import jax
import jax.numpy as jnp
from jax.experimental import pallas as pl

N = 16384
ND = N // 4
ROWS = 128
COLS = 128
RADIUS2 = np.float32(0.2) ** 2
EINV = np.float32(np.exp(np.float32(-1.0)))
BR = 256


def _fps_kernel(x0_ref, x1_ref, x2_ref, cent_ref):
    x0 = x0_ref[...]
    x1 = x1_ref[...]
    x2 = x2_ref[...]
    lin = (jax.lax.broadcasted_iota(jnp.int32, (ROWS, COLS), 0) * COLS
           + jax.lax.broadcasted_iota(jnp.int32, (ROWS, COLS), 1))
    lane = jax.lax.broadcasted_iota(jnp.int32, (1, COLS), 1)

    def body(i, carry):
        dist, j_prev = carry
        jr = j_prev // COLS
        jc = j_prev % COLS
        msk = lane == jc
        p0 = jnp.sum(jnp.where(msk, x0_ref[pl.ds(jr, 1), :], 0.0))
        p1 = jnp.sum(jnp.where(msk, x1_ref[pl.ds(jr, 1), :], 0.0))
        p2 = jnp.sum(jnp.where(msk, x2_ref[pl.ds(jr, 1), :], 0.0))
        cent_ref[pl.ds(i - 1, 1), :] = jnp.concatenate(
            [p0.reshape(1, 1), p1.reshape(1, 1), p2.reshape(1, 1)], axis=1)
        d0 = x0 - p0
        d1 = x1 - p1
        d2c = x2 - p2
        d_new = (d0 * d0 + d2c * d2c) + d1 * d1
        dist = jnp.minimum(dist, d_new)
        m = jnp.max(dist)
        j = jnp.min(jnp.where(dist == m, lin, jnp.int32(N)))
        return dist, j

    dist0 = jnp.full((ROWS, COLS), 1e30, dtype=jnp.float32)
    jax.lax.fori_loop(1, ND + 1, body, (dist0, jnp.int32(0)))


def _group_kernel(xb_ref, ct_ref, s_ref):
    xb = xb_ref[...]
    ct = ct_ref[...]
    x2 = jnp.sum(xb * xb, axis=1, keepdims=True)
    c2 = jnp.sum(ct * ct, axis=0, keepdims=True)
    d2 = x2 + c2 - 2.0 * jnp.dot(xb, ct, preferred_element_type=jnp.float32)
    mask = d2 < RADIUS2
    k = jnp.sum(jnp.where(mask, 1.0, 0.0), axis=1, keepdims=True)
    has = k > 0.0
    denom = jnp.where(has, k + (np.float32(ND) - k) * EINV, np.float32(ND))
    hi = 1.0 / denom
    lo = jnp.where(has, EINV, np.float32(1.0)) / denom
    s_ref[...] = jnp.where(mask, hi, lo)


def kernel(x):
    xt = x.T.reshape(3, ROWS, COLS)
    cents = pl.pallas_call(
        _fps_kernel,
        out_shape=jax.ShapeDtypeStruct((ND, 3), jnp.float32),
    )(xt[0], xt[1], xt[2])

    ct = cents.T
    s = pl.pallas_call(
        _group_kernel,
        grid=(N // BR,),
        in_specs=[
            pl.BlockSpec((BR, 3), lambda i: (i, 0)),
            pl.BlockSpec((3, ND), lambda i: (0, 0)),
        ],
        out_specs=pl.BlockSpec((BR, ND), lambda i: (i, 0)),
        out_shape=jax.ShapeDtypeStruct((N, ND), jnp.float32),
    )(x, ct)
    return (cents, s)


# speedup vs baseline: 41.8802x; 2.2347x over previous
import numpy as np
import jax
import jax.numpy as jnp
from jax.experimental import pallas as pl
from jax.experimental.pallas import tpu as pltpu

N = 16384
ND = N // 4
COLS = 128
RADIUS2 = np.float32(0.2 ** 2)
EINV = np.float32(np.exp(np.float32(-1.0)))
BR = 1024


T = 16
S = 8


def _tree_sum(a):
    while a.shape[0] > 1:
        half = a.shape[0] // 2
        a = a[:half] + a[half:]
    return a[0]


def _tree_max(a):
    while a.shape[0] > 1:
        half = a.shape[0] // 2
        a = jnp.maximum(a[:half], a[half:])
    return a[0]


def _fps_kernel(x0_ref, x1_ref, x2_ref, cent_ref, dist_ref):
    p0_init = x0_ref[0, 0:1, 0:1]
    p1_init = x1_ref[0, 0:1, 0:1]
    p2_init = x2_ref[0, 0:1, 0:1]
    cent_ref[0:1, :] = jnp.concatenate([p0_init, p1_init, p2_init], axis=1)
    dist_ref[...] = jnp.full((T, S, COLS), 1e30, dtype=jnp.float32)
    zsl = jnp.zeros((S, COLS), dtype=jnp.float32)

    def _bfly(s, op):
        s = op(s, pltpu.roll(s, 4, 0))
        s = op(s, pltpu.roll(s, 2, 0))
        return op(s, pltpu.roll(s, 1, 0))

    def _bcast_sum(w):
        return jnp.sum(_bfly(_tree_sum(w), jnp.add), axis=1, keepdims=True)

    def _dist_update(p0, p1, p2):
        x0 = x0_ref[...]
        x1 = x1_ref[...]
        x2 = x2_ref[...]
        d0 = x0 - p0
        d1 = x1 - p1
        d2 = x2 - p2
        d_new = (d0 * d0 + d2 * d2) + d1 * d1
        dist = jnp.minimum(dist_ref[...], d_new)
        dist_ref[...] = dist
        return dist, x0, x1, x2

    def body(i, carry):
        p0, p1, p2, tie = carry
        dist, x0, x1, x2 = _dist_update(p0[None], p1[None], p2[None])
        m1 = _bfly(_tree_max(dist), jnp.maximum)
        m = jnp.max(m1, axis=1, keepdims=True)
        hp = dist == m1[None]
        c0 = _bfly(_tree_sum(jnp.where(hp, x0, 0.0)), jnp.add)
        c1 = _bfly(_tree_sum(jnp.where(hp, x1, 0.0)), jnp.add)
        c2 = _bfly(_tree_sum(jnp.where(hp, x2, 0.0)), jnp.add)
        cc = _bfly(_tree_sum(jnp.where(hp, 1.0, 0.0)), jnp.add)
        h2 = m1 == m
        q0 = jnp.sum(jnp.where(h2, c0, 0.0), axis=1, keepdims=True)
        q1 = jnp.sum(jnp.where(h2, c1, 0.0), axis=1, keepdims=True)
        q2 = jnp.sum(jnp.where(h2, c2, 0.0), axis=1, keepdims=True)
        wc = jnp.sum(jnp.where(h2, cc, 0.0), axis=1, keepdims=True)
        tie = jnp.maximum(tie, wc)
        cent_ref[pl.ds(i, 1), :] = jnp.concatenate(
            [q0[0:1], q1[0:1], q2[0:1]], axis=1)
        zf = jnp.zeros((S, COLS), jnp.float32)
        return q0 + zf, q1 + zf, q2 + zf, tie

    p0b = p0_init + zsl
    p1b = p1_init + zsl
    p2b = p2_init + zsl
    out = jax.lax.fori_loop(
        1, ND, body, (p0b, p1b, p2b, jnp.zeros((S, 1), jnp.float32)))

    def exact_pass(_):
        lin = (jax.lax.broadcasted_iota(jnp.int32, (T, S, COLS), 0) * (S * COLS)
               + jax.lax.broadcasted_iota(jnp.int32, (T, S, COLS), 1) * COLS
               + jax.lax.broadcasted_iota(jnp.int32, (T, S, COLS), 2))
        dist_ref[...] = jnp.full((T, S, COLS), 1e30, dtype=jnp.float32)

        def ebody(i, carry):
            p0, p1, p2 = carry
            dist, x0, x1, x2 = _dist_update(p0, p1, p2)
            m = jnp.max(dist, axis=(0, 1, 2), keepdims=True)
            h = dist == m
            mn = jnp.min(jnp.where(h, lin, jnp.int32(N)),
                         axis=(0, 1, 2), keepdims=True)
            first = lin == mn
            q0 = jnp.sum(jnp.where(first, x0, 0.0), axis=(0, 1, 2),
                         keepdims=True)
            q1 = jnp.sum(jnp.where(first, x1, 0.0), axis=(0, 1, 2),
                         keepdims=True)
            q2 = jnp.sum(jnp.where(first, x2, 0.0), axis=(0, 1, 2),
                         keepdims=True)
            cent_ref[pl.ds(i, 1), :] = jnp.concatenate(
                [q0[0], q1[0], q2[0]], axis=1)
            return q0, q1, q2

        jax.lax.fori_loop(
            1, ND, ebody,
            (x0_ref[0, 0:1, 0:1][None], x1_ref[0, 0:1, 0:1][None],
             x2_ref[0, 0:1, 0:1][None]))
        return 0

    jax.lax.cond(jnp.max(out[3]) > 1.0, exact_pass, lambda _: 0, None)


def _group_kernel(xb_ref, ct_ref, s_ref):
    xb = xb_ref[...]
    ct = ct_ref[...]
    x2 = jnp.sum(xb * xb, axis=1, keepdims=True)
    c2 = jnp.sum(ct * ct, axis=0, keepdims=True)
    d2 = x2 + c2 - 2.0 * jnp.dot(xb, ct, preferred_element_type=jnp.float32)
    mask = d2 < RADIUS2
    k = jnp.sum(jnp.where(mask, 1.0, 0.0), axis=1, keepdims=True)
    has = k > 0.0
    denom = jnp.where(has, k + (np.float32(ND) - k) * EINV, np.float32(ND))
    hi = 1.0 / denom
    lo = jnp.where(has, EINV, np.float32(1.0)) / denom
    s_ref[...] = jnp.where(mask, hi, lo)


def kernel(x):
    xt = x.T.reshape(3, T, S, COLS)
    cents = pl.pallas_call(
        _fps_kernel,
        out_shape=jax.ShapeDtypeStruct((ND, 3), jnp.float32),
        scratch_shapes=[pltpu.VMEM((T, S, COLS), jnp.float32)],
    )(xt[0], xt[1], xt[2])

    ct = cents.T
    s = pl.pallas_call(
        _group_kernel,
        grid=(N // BR,),
        in_specs=[
            pl.BlockSpec((BR, 3), lambda i: (i, 0)),
            pl.BlockSpec((3, ND), lambda i: (0, 0)),
        ],
        out_specs=pl.BlockSpec((BR, ND), lambda i: (i, 0)),
        out_shape=jax.ShapeDtypeStruct((N, ND), jnp.float32),
    )(x, ct)
    return (cents, s)
